# Initial kernel scaffold; baseline (speedup 1.0000x reference)
import jax
import jax.numpy as jnp
from jax.experimental import pallas as pl


def kernel(x, edge_index, Wn0, Ws0, b0, Wn1, Ws1, b1, W_out, b_out):
    raise NotImplementedError("write your pallas kernel here")


# speedup vs baseline: 6.7181x; 6.7181x over previous
import functools

import jax
import jax.numpy as jnp
from jax import lax
from jax.experimental import pallas as pl
from jax.experimental.pallas import tpu as pltpu
from jax.experimental.pallas import tpu_sc as plsc

_NC = 2
_NS = 16
_NW = _NC * _NS
_CH = 80


def _sc_agg(y, src_r, dst_r, n_pad, compute_deg):
    n, h = y.shape
    nchunk = src_r.shape[1]
    rpt = n_pad // _NS
    assert rpt % _CH == 0

    mesh = plsc.VectorSubcoreMesh(
        core_axis_name="c", subcore_axis_name="s",
        num_cores=_NC, num_subcores=_NS)

    out_type = [jax.ShapeDtypeStruct((_NC, n_pad, h), jnp.float32)]
    scratch = [
        pltpu.VMEM((nchunk, _CH), jnp.int32),
        pltpu.VMEM((nchunk, _CH), jnp.int32),
        pltpu.VMEM((_CH, h), jnp.float32),
        pltpu.VMEM_SHARED((n_pad, h), jnp.float32),
        pltpu.SemaphoreType.DMA,
        pltpu.SemaphoreType.DMA,
    ]
    if compute_deg:
        out_type.append(jax.ShapeDtypeStruct((_NC, n_pad), jnp.float32))
        scratch += [
            pltpu.VMEM((_CH,), jnp.float32),
            pltpu.VMEM((rpt,), jnp.float32),
            pltpu.VMEM_SHARED((n_pad,), jnp.float32),
            pltpu.SemaphoreType.DMA,
        ]

    def body(y_hbm, src_hbm, dst_hbm, agg_out, *rest):
        if compute_deg:
            (deg_out, src_v, dst_v, rows_v, agg_sh, gsem, ssem,
             ones_v, zvec_v, deg_sh, dsem) = rest
        else:
            (src_v, dst_v, rows_v, agg_sh, gsem, ssem) = rest

        cid = lax.axis_index("c")
        sid = lax.axis_index("s")
        wid = sid * _NC + cid

        z16 = jnp.zeros((16,), jnp.float32)

        def zrow(i, _):
            def zcol(k, _):
                rows_v[i, pl.ds(k * 16, 16)] = z16
                return 0
            return lax.fori_loop(0, h // 16, zcol, 0)
        lax.fori_loop(0, _CH, zrow, 0)
        for r in range(rpt // _CH):
            pltpu.sync_copy(rows_v, agg_sh.at[pl.ds(sid * rpt + r * _CH, _CH)])

        if compute_deg:
            o16 = jnp.ones((16,), jnp.float32)
            def fones(i, _):
                ones_v[pl.ds(i * 16, 16)] = o16
                return 0
            lax.fori_loop(0, _CH // 16, fones, 0)
            def fz(i, _):
                zvec_v[pl.ds(i * 16, 16)] = z16
                return 0
            lax.fori_loop(0, rpt // 16, fz, 0)
            pltpu.sync_copy(zvec_v, deg_sh.at[pl.ds(sid * rpt, rpt)])

        pltpu.sync_copy(src_hbm.at[wid], src_v)
        pltpu.sync_copy(dst_hbm.at[wid], dst_v)

        plsc.subcore_barrier()

        def chunk(j, _):
            pltpu.async_copy(y_hbm.at[src_v.at[j]], rows_v, gsem).wait()
            pltpu.async_copy(rows_v, agg_sh.at[dst_v.at[j]], ssem,
                             add=True).wait()
            if compute_deg:
                pltpu.async_copy(ones_v, deg_sh.at[dst_v.at[j]], dsem,
                                 add=True).wait()
            return 0
        lax.fori_loop(0, nchunk, chunk, 0)

        plsc.subcore_barrier()

        pltpu.sync_copy(agg_sh.at[pl.ds(sid * rpt, rpt)],
                        agg_out.at[cid, pl.ds(sid * rpt, rpt)])
        if compute_deg:
            pltpu.sync_copy(deg_sh.at[pl.ds(sid * rpt, rpt)],
                            deg_out.at[cid, pl.ds(sid * rpt, rpt)])

    return pl.kernel(body, out_type=out_type, mesh=mesh,
                     scratch_types=scratch)(y, src_r, dst_r)


def _tc_transform(x, Wn, Ws, blk):
    n, d = x.shape
    h = Wn.shape[1]

    def body(x_ref, wn_ref, ws_ref, y_ref, z_ref):
        xb = x_ref[...]
        y_ref[...] = jnp.dot(xb, wn_ref[...], preferred_element_type=jnp.float32)
        z_ref[...] = jnp.dot(xb, ws_ref[...], preferred_element_type=jnp.float32)

    return pl.pallas_call(
        body,
        grid=(n // blk,),
        in_specs=[
            pl.BlockSpec((blk, d), lambda i: (i, 0)),
            pl.BlockSpec((d, h), lambda i: (0, 0)),
            pl.BlockSpec((d, h), lambda i: (0, 0)),
        ],
        out_specs=[
            pl.BlockSpec((blk, h), lambda i: (i, 0)),
            pl.BlockSpec((blk, h), lambda i: (i, 0)),
        ],
        out_shape=[
            jax.ShapeDtypeStruct((n, h), jnp.float32),
            jax.ShapeDtypeStruct((n, h), jnp.float32),
        ],
    )(x, Wn, Ws)


def _tc_mid(agg, deg3, z, b, Wn, Ws, n, blk):
    h = z.shape[1]

    def body(agg_ref, deg_ref, z_ref, b_ref, wn_ref, ws_ref, y_ref, z_out_ref):
        aggs = agg_ref[0] + agg_ref[1]
        degc = jnp.clip(deg_ref[0] + deg_ref[1], 1.0)
        hb = jax.nn.relu(aggs / degc + z_ref[...] + b_ref[...])
        y_ref[...] = jnp.dot(hb, wn_ref[...], preferred_element_type=jnp.float32)
        z_out_ref[...] = jnp.dot(hb, ws_ref[...], preferred_element_type=jnp.float32)

    return pl.pallas_call(
        body,
        grid=(n // blk,),
        in_specs=[
            pl.BlockSpec((_NC, blk, h), lambda i: (0, i, 0)),
            pl.BlockSpec((_NC, blk, 1), lambda i: (0, i, 0)),
            pl.BlockSpec((blk, h), lambda i: (i, 0)),
            pl.BlockSpec((1, h), lambda i: (0, 0)),
            pl.BlockSpec((h, h), lambda i: (0, 0)),
            pl.BlockSpec((h, h), lambda i: (0, 0)),
        ],
        out_specs=[
            pl.BlockSpec((blk, h), lambda i: (i, 0)),
            pl.BlockSpec((blk, h), lambda i: (i, 0)),
        ],
        out_shape=[
            jax.ShapeDtypeStruct((n, h), jnp.float32),
            jax.ShapeDtypeStruct((n, h), jnp.float32),
        ],
    )(agg, deg3, z, b.reshape(1, h), Wn, Ws)


def _tc_final(agg, deg3, z, b, W_out, b_out, n, blk):
    h = z.shape[1]
    nblk = n // blk

    def body(agg_ref, deg_ref, z_ref, b_ref, wo_ref, bo_ref, out_ref, acc_ref):
        i = pl.program_id(0)
        aggs = agg_ref[0] + agg_ref[1]
        degc = jnp.clip(deg_ref[0] + deg_ref[1], 1.0)
        hb = jax.nn.relu(aggs / degc + z_ref[...] + b_ref[...])
        psum = jnp.sum(hb, axis=0, keepdims=True)

        @pl.when(i == 0)
        def _():
            acc_ref[...] = jnp.zeros_like(acc_ref)
        acc_ref[...] += psum

        @pl.when(i == nblk - 1)
        def _():
            pooled = acc_ref[...] * (1.0 / n)
            out_ref[...] = jnp.dot(pooled, wo_ref[...],
                                   preferred_element_type=jnp.float32) + bo_ref[...]

    return pl.pallas_call(
        body,
        grid=(nblk,),
        in_specs=[
            pl.BlockSpec((_NC, blk, h), lambda i: (0, i, 0)),
            pl.BlockSpec((_NC, blk, 1), lambda i: (0, i, 0)),
            pl.BlockSpec((blk, h), lambda i: (i, 0)),
            pl.BlockSpec((1, h), lambda i: (0, 0)),
            pl.BlockSpec((h, 1), lambda i: (0, 0)),
            pl.BlockSpec((1, 1), lambda i: (0, 0)),
        ],
        out_specs=pl.BlockSpec((1, 1), lambda i: (0, 0)),
        out_shape=jax.ShapeDtypeStruct((1, 1), jnp.float32),
        scratch_shapes=[pltpu.VMEM((1, h), jnp.float32)],
    )(agg, deg3, z, b.reshape(1, h), W_out, b_out.reshape(1, 1))


def kernel(x, edge_index, Wn0, Ws0, b0, Wn1, Ws1, b1, W_out, b_out):
    n, d = x.shape
    h = Wn0.shape[1]
    e = edge_index.shape[1]

    n_pad = ((n + _NS * _CH - 1) // (_NS * _CH)) * (_NS * _CH)

    src = edge_index[0].astype(jnp.int32)
    dst = edge_index[1].astype(jnp.int32)

    e_pad = ((e + _NW * _CH - 1) // (_NW * _CH)) * (_NW * _CH)
    if e_pad != e:
        src = jnp.pad(src, (0, e_pad - e))
        dst = jnp.pad(dst, (0, e_pad - e), constant_values=n)
    nchunk = e_pad // (_NW * _CH)
    src_r = src.reshape(_NW, nchunk, _CH)
    dst_r = dst.reshape(_NW, nchunk, _CH)

    blk = max(b for b in range(8, 513, 8) if n % b == 0)

    y0, z0 = _tc_transform(x, Wn0, Ws0, blk)
    agg0, deg = _sc_agg(y0, src_r, dst_r, n_pad, compute_deg=True)
    deg3 = deg.reshape(_NC, n_pad, 1)
    y1, z1 = _tc_mid(agg0, deg3, z0, b0, Wn1, Ws1, n, blk)
    (agg1,) = _sc_agg(y1, src_r, dst_r, n_pad, compute_deg=False)
    out = _tc_final(agg1, deg3, z1, b1, W_out, b_out, n, blk)
    return out.reshape(1)
